# Initial kernel scaffold; baseline (speedup 1.0000x reference)
import jax
import jax.numpy as jnp
from jax.experimental import pallas as pl


def kernel(nodes, senders, receivers, W_l, b_l, W_r, b_r, W_a, b_a):
    raise NotImplementedError("write your pallas kernel here")


# speedup vs baseline: 1.0002x; 1.0002x over previous
import jax
import jax.numpy as jnp
from jax.experimental import pallas as pl
from jax.experimental.pallas import tpu as pltpu

N = 10000
E = 320000
DF = 128
H = 4
D = 32

_ROWS_PER_BLOCK = 1000


def _proj_body(nodes_ref, w_ref, b_ref, out_ref):
    out_ref[...] = (
        jnp.dot(nodes_ref[...], w_ref[...], preferred_element_type=jnp.float32)
        + b_ref[...]
    )


def _project(nodes, W, b):
    n = nodes.shape[0]
    k = W.shape[1]
    grid = n // _ROWS_PER_BLOCK
    return pl.pallas_call(
        _proj_body,
        grid=(grid,),
        in_specs=[
            pl.BlockSpec((_ROWS_PER_BLOCK, DF), lambda i: (i, 0)),
            pl.BlockSpec((DF, k), lambda i: (0, 0)),
            pl.BlockSpec((1, k), lambda i: (0, 0)),
        ],
        out_specs=pl.BlockSpec((_ROWS_PER_BLOCK, k), lambda i: (i, 0)),
        out_shape=jax.ShapeDtypeStruct((n, k), jnp.float32),
    )(nodes, W, b.reshape(1, k))


def kernel(nodes, senders, receivers, W_l, b_l, W_r, b_r, W_a, b_a):
    W = jnp.concatenate([W_l, W_r], axis=1)
    b = jnp.concatenate([b_l, b_r], axis=0)
    q = _project(nodes, W, b)
    ql = q[:, : H * D].reshape(-1, H, D)
    qr = q[:, H * D :].reshape(-1, H, D)
    sent = ql[senders]
    recv = qr[receivers]
    z = jax.nn.leaky_relu(sent + recv, negative_slope=0.2)
    logits = z @ W_a + b_a
    maxs = jax.ops.segment_max(logits, receivers, num_segments=N)
    maxs = jnp.where(jnp.isfinite(maxs), maxs, 0.0)
    shifted = logits - jax.lax.stop_gradient(maxs[receivers])
    unnorm = jnp.exp(shifted)
    denom = jax.ops.segment_sum(unnorm, receivers, num_segments=N)
    alpha = unnorm / denom[receivers]
    out = sent * alpha
    out = jax.ops.segment_sum(out, receivers, num_segments=N)
    return out.reshape(N, H * D)


# speedup vs baseline: 37.3027x; 37.2967x over previous
import functools

import jax
import jax.numpy as jnp
from jax import lax
from jax.experimental import pallas as pl
from jax.experimental.pallas import tpu as pltpu
from jax.experimental.pallas import tpu_sc as plsc

NN = 10000
EE = 320000
DFEAT = 128
NH = 4
DH = 32

NC = 2
NS = 16
NW = NC * NS
BLK = 128
NEG = -1e38

_ROWS_PER_BLOCK = 1000


def _proj_body(nodes_ref, wl_ref, bl_ref, wr_ref, br_ref, ql_ref, qr_ref):
    x = nodes_ref[...]
    ql_ref[...] = jnp.dot(x, wl_ref[...], preferred_element_type=jnp.float32) + bl_ref[...]
    qr_ref[...] = jnp.dot(x, wr_ref[...], preferred_element_type=jnp.float32) + br_ref[...]


def _project(nodes, W_l, b_l, W_r, b_r):
    n = nodes.shape[0]
    k = W_l.shape[1]
    grid = n // _ROWS_PER_BLOCK
    return pl.pallas_call(
        _proj_body,
        grid=(grid,),
        in_specs=[
            pl.BlockSpec((_ROWS_PER_BLOCK, DFEAT), lambda i: (i, 0)),
            pl.BlockSpec((DFEAT, k), lambda i: (0, 0)),
            pl.BlockSpec((1, k), lambda i: (0, 0)),
            pl.BlockSpec((DFEAT, k), lambda i: (0, 0)),
            pl.BlockSpec((1, k), lambda i: (0, 0)),
        ],
        out_specs=[
            pl.BlockSpec((_ROWS_PER_BLOCK, k), lambda i: (i, 0)),
            pl.BlockSpec((_ROWS_PER_BLOCK, k), lambda i: (i, 0)),
        ],
        out_shape=[
            jax.ShapeDtypeStruct((n, k), jnp.float32),
            jax.ShapeDtypeStruct((n, k), jnp.float32),
        ],
    )(nodes, W_l, b_l.reshape(1, k), W_r, b_r.reshape(1, k))


def _edge_body(ql_h, qr_h, snd_h, rcv_h, nb_h, eb_h, wa_h, out_h,
               sidx, ridx, rsc, sent, recv, rowbuf, zrow, nb_v, eb_v, wa_v,
               sem_s, sem_r):
    wid = lax.axis_index("c") * NS + lax.axis_index("s")
    pltpu.sync_copy(nb_h, nb_v)
    pltpu.sync_copy(eb_h, eb_v)
    pltpu.sync_copy(wa_h, wa_v)
    nbv = nb_v[pl.ds(wid, 16)]
    ebv = eb_v[pl.ds(wid, 16)]
    n_lo = nbv[0]
    n_hi = nbv[1]
    e0 = ebv[0]
    e1 = ebv[1]
    wa = [wa_v[pl.ds(16 * j, 16)] for j in range(2)]

    zv = jnp.zeros((16,), jnp.float32)
    for j in range(8):
        zrow[pl.ds(16 * j, 16)] = zv

    def write_row(node, dvs, avs):
        for j in range(8):
            rowbuf[pl.ds(16 * j, 16)] = avs[j] / dvs[j // 2]
        pltpu.sync_copy(rowbuf, out_h.at[node])

    def zero_rows(lo, hi):
        def zbody(g, c):
            pltpu.sync_copy(zrow, out_h.at[g])
            return c
        lax.fori_loop(lo, hi, zbody, 0)

    base0 = (e0 // BLK) * BLK
    nblk = (e1 - base0 + BLK - 1) // BLK

    negv = jnp.full((16,), NEG, jnp.float32)

    def blk_body(blk, carry):
        bstart = base0 + blk * BLK
        pltpu.sync_copy(snd_h.at[pl.ds(bstart, BLK)], sidx)
        pltpu.sync_copy(rcv_h.at[pl.ds(bstart, BLK)], ridx)
        pltpu.sync_copy(rcv_h.at[pl.ds(bstart, BLK)], rsc.at[pl.ds(0, BLK)])
        cp_s = pltpu.async_copy(ql_h.at[sidx], sent, sem_s)
        cp_r = pltpu.async_copy(qr_h.at[ridx], recv, sem_r)
        cp_s.wait()
        cp_r.wait()
        lo = jnp.maximum(e0, bstart)
        hi = jnp.minimum(e1, bstart + BLK)

        def edge_body(e, ec):
            cur = ec[0]
            ms = list(ec[1:5])
            ds = list(ec[5:9])
            avs = list(ec[9:17])
            i = e - bstart
            r = rsc[pl.ds(i, 16)][0]
            is_new = r != cur

            @pl.when(is_new)
            def _():
                @pl.when(cur >= n_lo)
                def _():
                    write_row(cur, ds, avs)
                zero_rows(cur + 1, r)

            ms = [jnp.where(is_new, negv, m) for m in ms]
            ds = [jnp.where(is_new, jnp.zeros((16,), jnp.float32), d) for d in ds]
            avs = [jnp.where(is_new, jnp.zeros((16,), jnp.float32), a) for a in avs]

            svs = [sent[i, pl.ds(16 * j, 16)] for j in range(8)]
            rvs = [recv[i, pl.ds(16 * j, 16)] for j in range(8)]
            pvs = []
            for j in range(8):
                z = svs[j] + rvs[j]
                z = jnp.maximum(z, 0.2 * z)
                pvs.append(z * wa[j % 2])
            for h in range(4):
                lsum = jnp.sum(pvs[2 * h] + pvs[2 * h + 1])
                lv = jnp.full((16,), lsum)
                mn = jnp.maximum(ms[h], lv)
                sc = jnp.exp(ms[h] - mn)
                el = jnp.exp(lv - mn)
                ds[h] = ds[h] * sc + el
                avs[2 * h] = avs[2 * h] * sc + el * svs[2 * h]
                avs[2 * h + 1] = avs[2 * h + 1] * sc + el * svs[2 * h + 1]
                ms[h] = mn
            return (r, *ms, *ds, *avs)

        return lax.fori_loop(lo, hi, edge_body, carry)

    zvec = jnp.zeros((16,), jnp.float32)
    init = (n_lo - 1,
            negv, negv, negv, negv,
            zvec, zvec, zvec, zvec,
            zvec, zvec, zvec, zvec, zvec, zvec, zvec, zvec)
    fin = lax.fori_loop(0, nblk, blk_body, init)
    cur = fin[0]
    ds = list(fin[5:9])
    avs = list(fin[9:17])

    @pl.when(cur >= n_lo)
    def _():
        write_row(cur, ds, avs)

    zero_rows(cur + 1, n_hi)


@functools.partial(
    pl.kernel,
    out_type=jax.ShapeDtypeStruct((NN, DFEAT), jnp.float32),
    mesh=plsc.VectorSubcoreMesh(core_axis_name="c", subcore_axis_name="s",
                                num_cores=NC, num_subcores=NS),
    scratch_types=[
        pltpu.VMEM((BLK,), jnp.int32),
        pltpu.VMEM((BLK,), jnp.int32),
        pltpu.VMEM((BLK + 16,), jnp.int32),
        pltpu.VMEM((BLK, DFEAT), jnp.float32),
        pltpu.VMEM((BLK, DFEAT), jnp.float32),
        pltpu.VMEM((DFEAT,), jnp.float32),
        pltpu.VMEM((DFEAT,), jnp.float32),
        pltpu.VMEM((48,), jnp.int32),
        pltpu.VMEM((48,), jnp.int32),
        pltpu.VMEM((32,), jnp.float32),
        pltpu.SemaphoreType.DMA,
        pltpu.SemaphoreType.DMA,
    ],
    compiler_params=pltpu.CompilerParams(needs_layout_passes=False),
)
def _edge_kernel(*refs):
    _edge_body(*refs)


def kernel(nodes, senders, receivers, W_l, b_l, W_r, b_r, W_a, b_a):
    ql, qr = _project(nodes, W_l, b_l, W_r, b_r)
    pos = jnp.arange(1, NW, dtype=jnp.int32) * (EE // NW)
    nb_mid = receivers[pos]
    node_bounds = jnp.concatenate([
        jnp.zeros((1,), jnp.int32), nb_mid,
        jnp.full((1,), NN, jnp.int32),
    ])
    edge_bounds = jnp.searchsorted(receivers, node_bounds).astype(jnp.int32)
    nb_pad = jnp.zeros((48,), jnp.int32).at[:NW + 1].set(node_bounds)
    eb_pad = jnp.zeros((48,), jnp.int32).at[:NW + 1].set(edge_bounds)
    wa = W_a.reshape(DH)
    out = _edge_kernel(ql, qr, senders, receivers, nb_pad, eb_pad, wa)
    return out


# speedup vs baseline: 47.2545x; 1.2668x over previous
import functools

import jax
import jax.numpy as jnp
from jax import lax
from jax.experimental import pallas as pl
from jax.experimental.pallas import tpu as pltpu
from jax.experimental.pallas import tpu_sc as plsc

NN = 10000
EE = 320000
DFEAT = 128
NH = 4
DH = 32

NC = 2
NS = 16
NW = NC * NS
BLK = 128
NEG = -1e38

_ROWS_PER_BLOCK = 1000


def _proj_body(nodes_ref, wl_ref, bl_ref, wr_ref, br_ref, ql_ref, qr_ref):
    x = nodes_ref[...]
    ql_ref[...] = jnp.dot(x, wl_ref[...], preferred_element_type=jnp.float32) + bl_ref[...]
    qr_ref[...] = jnp.dot(x, wr_ref[...], preferred_element_type=jnp.float32) + br_ref[...]


def _project(nodes, W_l, b_l, W_r, b_r):
    n = nodes.shape[0]
    k = W_l.shape[1]
    grid = n // _ROWS_PER_BLOCK
    return pl.pallas_call(
        _proj_body,
        grid=(grid,),
        in_specs=[
            pl.BlockSpec((_ROWS_PER_BLOCK, DFEAT), lambda i: (i, 0)),
            pl.BlockSpec((DFEAT, k), lambda i: (0, 0)),
            pl.BlockSpec((1, k), lambda i: (0, 0)),
            pl.BlockSpec((DFEAT, k), lambda i: (0, 0)),
            pl.BlockSpec((1, k), lambda i: (0, 0)),
        ],
        out_specs=[
            pl.BlockSpec((_ROWS_PER_BLOCK, k), lambda i: (i, 0)),
            pl.BlockSpec((_ROWS_PER_BLOCK, k), lambda i: (i, 0)),
        ],
        out_shape=[
            jax.ShapeDtypeStruct((n, k), jnp.float32),
            jax.ShapeDtypeStruct((n, k), jnp.float32),
        ],
    )(nodes, W_l, b_l.reshape(1, k), W_r, b_r.reshape(1, k))


def _edge_body(ql_h, qr_h, snd_h, rcv_h, nb_h, eb_h, wa_h, out_h,
               sidx, ridx, rsc, sent, recv, rowbuf, zrow, nb_v, eb_v, wa_v,
               sem_s, sem_r):
    wid = lax.axis_index("c") * NS + lax.axis_index("s")
    pltpu.sync_copy(nb_h, nb_v)
    pltpu.sync_copy(eb_h, eb_v)
    pltpu.sync_copy(wa_h, wa_v)
    nbv = nb_v[pl.ds(wid, 16)]
    ebv = eb_v[pl.ds(wid, 16)]
    n_lo = nbv[0]
    n_hi = nbv[1]
    e0 = ebv[0]
    e1 = ebv[1]
    wa = [wa_v[pl.ds(16 * j, 16)] for j in range(2)]

    zv = jnp.zeros((16,), jnp.float32)
    for j in range(8):
        zrow[pl.ds(16 * j, 16)] = zv

    def write_row(node, dvs, avs):
        for j in range(8):
            rowbuf[pl.ds(16 * j, 16)] = avs[j] / dvs[j // 2]
        pltpu.sync_copy(rowbuf, out_h.at[node])

    def zero_rows(lo, hi):
        def zbody(g, c):
            pltpu.sync_copy(zrow, out_h.at[g])
            return c
        lax.fori_loop(lo, hi, zbody, 0)

    base0 = (e0 // BLK) * BLK
    nblk = (e1 - base0 + BLK - 1) // BLK

    negv = jnp.full((16,), NEG, jnp.float32)

    def stage_block(b, par):
        bstart = base0 + b * BLK
        pltpu.sync_copy(snd_h.at[pl.ds(bstart, BLK)], sidx.at[par])
        pltpu.sync_copy(rcv_h.at[pl.ds(bstart, BLK)], ridx.at[par])
        pltpu.sync_copy(rcv_h.at[pl.ds(bstart, BLK)], rsc.at[par, pl.ds(0, BLK)])
        pltpu.async_copy(ql_h.at[sidx.at[par]], sent.at[par], sem_s.at[par])
        pltpu.async_copy(qr_h.at[ridx.at[par]], recv.at[par], sem_r.at[par])

    @pl.when(nblk > 0)
    def _():
        stage_block(jnp.int32(0), jnp.int32(0))

    def blk_body(blk, carry):
        bstart = base0 + blk * BLK
        par = lax.rem(blk, 2)

        @pl.when(blk + 1 < nblk)
        def _():
            stage_block(blk + 1, 1 - par)

        pltpu.make_async_copy(ql_h.at[sidx.at[par]], sent.at[par],
                              sem_s.at[par]).wait()
        pltpu.make_async_copy(qr_h.at[ridx.at[par]], recv.at[par],
                              sem_r.at[par]).wait()
        lo = jnp.maximum(e0, bstart)
        hi = jnp.minimum(e1, bstart + BLK)

        def edge_body(e, ec):
            cur = ec[0]
            ms = list(ec[1:5])
            ds = list(ec[5:9])
            avs = list(ec[9:17])
            i = e - bstart
            r = rsc[par, pl.ds(i, 16)][0]
            is_new = r != cur

            @pl.when(is_new)
            def _():
                @pl.when(cur >= n_lo)
                def _():
                    write_row(cur, ds, avs)
                zero_rows(cur + 1, r)

            ms = [jnp.where(is_new, negv, m) for m in ms]
            ds = [jnp.where(is_new, jnp.zeros((16,), jnp.float32), d) for d in ds]
            avs = [jnp.where(is_new, jnp.zeros((16,), jnp.float32), a) for a in avs]

            svs = [sent[par, i, pl.ds(16 * j, 16)] for j in range(8)]
            rvs = [recv[par, i, pl.ds(16 * j, 16)] for j in range(8)]
            pvs = []
            for j in range(8):
                z = svs[j] + rvs[j]
                z = jnp.maximum(z, 0.2 * z)
                pvs.append(z * wa[j % 2])
            for h in range(4):
                lsum = jnp.sum(pvs[2 * h] + pvs[2 * h + 1])
                lv = jnp.full((16,), lsum)
                mn = jnp.maximum(ms[h], lv)
                sc = jnp.exp(ms[h] - mn)
                el = jnp.exp(lv - mn)
                ds[h] = ds[h] * sc + el
                avs[2 * h] = avs[2 * h] * sc + el * svs[2 * h]
                avs[2 * h + 1] = avs[2 * h + 1] * sc + el * svs[2 * h + 1]
                ms[h] = mn
            return (r, *ms, *ds, *avs)

        return lax.fori_loop(lo, hi, edge_body, carry)

    zvec = jnp.zeros((16,), jnp.float32)
    init = (n_lo - 1,
            negv, negv, negv, negv,
            zvec, zvec, zvec, zvec,
            zvec, zvec, zvec, zvec, zvec, zvec, zvec, zvec)
    fin = lax.fori_loop(0, nblk, blk_body, init)
    cur = fin[0]
    ds = list(fin[5:9])
    avs = list(fin[9:17])

    @pl.when(cur >= n_lo)
    def _():
        write_row(cur, ds, avs)

    zero_rows(cur + 1, n_hi)


@functools.partial(
    pl.kernel,
    out_type=jax.ShapeDtypeStruct((NN, DFEAT), jnp.float32),
    mesh=plsc.VectorSubcoreMesh(core_axis_name="c", subcore_axis_name="s",
                                num_cores=NC, num_subcores=NS),
    scratch_types=[
        pltpu.VMEM((2, BLK), jnp.int32),
        pltpu.VMEM((2, BLK), jnp.int32),
        pltpu.VMEM((2, BLK + 16), jnp.int32),
        pltpu.VMEM((2, BLK, DFEAT), jnp.float32),
        pltpu.VMEM((2, BLK, DFEAT), jnp.float32),
        pltpu.VMEM((DFEAT,), jnp.float32),
        pltpu.VMEM((DFEAT,), jnp.float32),
        pltpu.VMEM((48,), jnp.int32),
        pltpu.VMEM((48,), jnp.int32),
        pltpu.VMEM((32,), jnp.float32),
        pltpu.SemaphoreType.DMA((2,)),
        pltpu.SemaphoreType.DMA((2,)),
    ],
    compiler_params=pltpu.CompilerParams(needs_layout_passes=False),
)
def _edge_kernel(*refs):
    _edge_body(*refs)


def kernel(nodes, senders, receivers, W_l, b_l, W_r, b_r, W_a, b_a):
    ql, qr = _project(nodes, W_l, b_l, W_r, b_r)
    pos = jnp.arange(1, NW, dtype=jnp.int32) * (EE // NW)
    nb_mid = receivers[pos]
    node_bounds = jnp.concatenate([
        jnp.zeros((1,), jnp.int32), nb_mid,
        jnp.full((1,), NN, jnp.int32),
    ])
    edge_bounds = jnp.searchsorted(receivers, node_bounds).astype(jnp.int32)
    nb_pad = jnp.zeros((48,), jnp.int32).at[:NW + 1].set(node_bounds)
    eb_pad = jnp.zeros((48,), jnp.int32).at[:NW + 1].set(edge_bounds)
    wa = W_a.reshape(DH)
    out = _edge_kernel(ql, qr, senders, receivers, nb_pad, eb_pad, wa)
    return out
